# baseline (device time: 134445 ns/iter reference)
import functools
import os

import jax
import jax.numpy as jnp
from jax import lax
from jax.experimental import pallas as pl
from jax.experimental.pallas import tpu as pltpu

MESH = pl.DeviceIdType.MESH
_COMPUTE_ONLY = bool(int(os.environ.get("KERNEL_COMPUTE_ONLY", "0")))
_NO_CID = bool(int(os.environ.get("KERNEL_NO_CID", "0")))
_FUSE_TLHS = bool(int(os.environ.get("KERNEL_FUSE_TLHS", "0")))


def kernel(x, dy):
    K, D = x.shape
    _, F = dy.shape
    G = F // 4
    H = D // 2
    T = 4
    TW = G // T
    HW = TW // 2
    U = 2 * T

    def body(x_hbm, dy_hbm, out_hbm, ld, xb, dyb, pk, zs, zr, sb, gx, gy,
             hx, hy, hd, cb, ld_sem, zs_s, zr_s, axs, axr, ays, ayr,
             bxs, bxr, bys, byr, ads, adr, sts, stc):
        mx = lax.axis_index("x")
        my = lax.axis_index("y")
        mz = lax.axis_index("z")
        g = 2 * mx + my
        gp = 2 * (1 - mx) + my
        hh = 2 * mx + (1 - my)
        hp = 2 * (1 - mx) + (1 - my)
        xpeer = (1 - mx, my, mz)
        ypeer = (mx, 1 - my, mz)
        zpeer = (mx, my, 1 - mz)
        dpeer = (1 - mx, 1 - my, mz)
        half0 = (1 - mz) * H
        half1 = mz * H

        load_plan = [("x", half0), ("x", half0 + TW), ("dy", 0), ("dy", 1),
                     ("x", half1), ("x", half1 + TW), ("dy", 2), ("dy", 3)]

        def start_load(c):
            kind, col = load_plan[c]
            src_col = col if kind == "x" else g * G + col * TW
            cp = pltpu.make_async_copy(
                (x_hbm if kind == "x" else dy_hbm).at[:, pl.ds(src_col, TW)],
                ld.at[c % 2], ld_sem.at[c % 2])
            cp.start()
            return cp

        pending = {0: start_load(0)}

        def finish_load(c):
            pending.pop(c).wait()
            if c + 1 < len(load_plan):
                pending[c + 1] = start_load(c + 1)
            kind, col = load_plan[c]
            val = ld[c % 2].astype(jnp.bfloat16)
            if kind == "x":
                xb[:, pl.ds(col, TW)] = val
            else:
                dyb[col % 2] = val

        if not _COMPUTE_ONLY and not _NO_CID:
            bar = pltpu.get_barrier_semaphore()
            for dev in (xpeer, ypeer, zpeer, dpeer):
                pl.semaphore_signal(bar, inc=1, device_id=dev,
                                    device_id_type=MESH)
            pl.semaphore_wait(bar, 4)

        dn = (((0,), (0,)), ((), ()))

        def send_mm(t):
            p = lax.dot_general(
                xb[:, pl.ds(half0, H)], dyb[t % 2], dn,
                preferred_element_type=jnp.float32).astype(jnp.bfloat16)
            rs = []
            for u in (2 * t, 2 * t + 1):
                zs[u] = p[:, (u % 2) * HW:(u % 2) * HW + HW]
                if not _COMPUTE_ONLY:
                    r = pltpu.make_async_remote_copy(
                        zs.at[u], zr.at[u], zs_s.at[u], zr_s.at[u],
                        device_id=zpeer, device_id_type=MESH)
                    r.start()
                    rs.append(r)
            return rs

        def keep_mm(t):
            pk[t] = lax.dot_general(
                xb[:, pl.ds(half1, H)], dyb[t % 2], dn,
                preferred_element_type=jnp.float32)

        zrd = {}
        finish_load(0)
        finish_load(1)
        finish_load(2)
        zrd[0] = send_mm(0)
        finish_load(3)
        zrd[1] = send_mm(1)
        finish_load(4)
        finish_load(5)
        keep_mm(0)
        keep_mm(1)
        finish_load(6)
        zrd[2] = send_mm(2)
        keep_mm(2)
        finish_load(7)
        zrd[3] = send_mm(3)
        keep_mm(3)

        stcp = [None, None]
        cb_uses = [0]

        def store_via_cb(val_bf16, out_col):
            slot = cb_uses[0] % 2
            if stcp[slot] is not None:
                stcp[slot].wait()
            cb[slot] = val_bf16.astype(jnp.float32)
            cp = pltpu.make_async_copy(
                cb.at[slot], out_hbm.at[:, pl.ds(out_col, TW)], stc.at[slot])
            cp.start()
            stcp[slot] = cp
            cb_uses[0] += 1

        axd, ayd, std = {}, {}, []
        for u in range(U):
            t, i = u // 2, u % 2
            if not _COMPUTE_ONLY:
                zrd[t][i].wait()
            s = pk[t][:, i * HW:i * HW + HW] + (
                (zs if _COMPUTE_ONLY else zr)[u].astype(jnp.float32))
            pk[t, :, i * HW:i * HW + HW] = s
            sb[u] = s.astype(jnp.bfloat16)
            if not _COMPUTE_ONLY:
                ax = pltpu.make_async_remote_copy(
                    sb.at[u], gx.at[u], axs.at[u], axr.at[u],
                    device_id=xpeer, device_id_type=MESH)
                ax.start()
                axd[u] = ax
                ay = pltpu.make_async_remote_copy(
                    sb.at[u], gy.at[u], ays.at[u], ayr.at[u],
                    device_id=ypeer, device_id_type=MESH)
                ay.start()
                ayd[u] = ay
                if u == U - 1:
                    ad = pltpu.make_async_remote_copy(
                        sb.at[pl.ds(U - 2, 2)], hd, ads, adr,
                        device_id=dpeer, device_id_type=MESH)
                    ad.start()
            if i == 1:
                st = pltpu.make_async_copy(
                    pk.at[t], out_hbm.at[:, pl.ds(g * G + t * TW, TW)],
                    sts.at[t])
                st.start()
                std.append(st)

        bxd, byd = [], []
        gxr = sb if _COMPUTE_ONLY else gx
        gyr = sb if _COMPUTE_ONLY else gy
        for t in range(T):
            if not _COMPUTE_ONLY:
                axd[2 * t + 1].wait()
                if t < T - 1:
                    by = pltpu.make_async_remote_copy(
                        gx.at[2 * t + 1], hy.at[t], bys.at[t], byr.at[t],
                        device_id=ypeer, device_id_type=MESH)
                    by.start()
                    byd.append(by)
                ayd[2 * t].wait()
                if t < T - 1:
                    bx = pltpu.make_async_remote_copy(
                        gy.at[2 * t], hx.at[t], bxs.at[t], bxr.at[t],
                        device_id=xpeer, device_id_type=MESH)
                    bx.start()
                    bxd.append(bx)
                axd[2 * t].wait()
                ayd[2 * t + 1].wait()
            store_via_cb(
                jnp.concatenate([gxr[2 * t], gxr[2 * t + 1]], axis=1),
                gp * G + t * TW)
            store_via_cb(
                jnp.concatenate([gyr[2 * t], gyr[2 * t + 1]], axis=1),
                hh * G + t * TW)

        for t in range(T):
            if _COMPUTE_ONLY:
                hx_t, hy_t = sb[2 * t], sb[2 * t + 1]
            elif t < T - 1:
                bxd[t].wait()
                byd[t].wait()
                hx_t, hy_t = hx[t], hy[t]
            else:
                ad.wait()
                hx_t, hy_t = hd[0], hd[1]
            store_via_cb(
                jnp.concatenate([hx_t, hy_t], axis=1), hp * G + t * TW)

        for st in std:
            st.wait()
        for cp in stcp:
            if cp is not None:
                cp.wait()

        if not _COMPUTE_ONLY:
            @functools.partial(pl.run_scoped,
                               sem2=pltpu.SemaphoreType.REGULAR)
            def _(sem2):
                for dev in (xpeer, ypeer, zpeer, dpeer):
                    pl.semaphore_signal(sem2, inc=1, device_id=dev,
                                        device_id_type=MESH)
                pl.semaphore_wait(sem2, 4)

    return pl.pallas_call(
        body,
        out_shape=jax.ShapeDtypeStruct((H, F), jnp.float32),
        in_specs=[
            pl.BlockSpec(memory_space=pl.ANY),
            pl.BlockSpec(memory_space=pl.ANY),
        ],
        out_specs=pl.BlockSpec(memory_space=pl.ANY),
        scratch_shapes=[
            pltpu.VMEM((2, K, TW), jnp.float32),
            pltpu.VMEM((K, D), jnp.bfloat16),
            pltpu.VMEM((2, K, TW), jnp.bfloat16),
            pltpu.VMEM((T, H, TW), jnp.float32),
            pltpu.VMEM((U, H, HW), jnp.bfloat16),
            pltpu.VMEM((U, H, HW), jnp.bfloat16),
            pltpu.VMEM((U, H, HW), jnp.bfloat16),
            pltpu.VMEM((U, H, HW), jnp.bfloat16),
            pltpu.VMEM((U, H, HW), jnp.bfloat16),
            pltpu.VMEM((T, H, HW), jnp.bfloat16),
            pltpu.VMEM((T, H, HW), jnp.bfloat16),
            pltpu.VMEM((2, H, HW), jnp.bfloat16),
            pltpu.VMEM((2, H, TW), jnp.float32),
            pltpu.SemaphoreType.DMA((2,)),
            pltpu.SemaphoreType.DMA((U,)),
            pltpu.SemaphoreType.DMA((U,)),
            pltpu.SemaphoreType.DMA((U,)),
            pltpu.SemaphoreType.DMA((U,)),
            pltpu.SemaphoreType.DMA((U,)),
            pltpu.SemaphoreType.DMA((U,)),
            pltpu.SemaphoreType.DMA((T,)),
            pltpu.SemaphoreType.DMA((T,)),
            pltpu.SemaphoreType.DMA((T,)),
            pltpu.SemaphoreType.DMA((T,)),
            pltpu.SemaphoreType.DMA,
            pltpu.SemaphoreType.DMA,
            pltpu.SemaphoreType.DMA((T,)),
            pltpu.SemaphoreType.DMA((2,)),
        ],
        compiler_params=pltpu.CompilerParams(
            collective_id=None if (_COMPUTE_ONLY or _NO_CID) else 0,
            vmem_limit_bytes=100 * 1024 * 1024,
            fuse_transposed_lhs_in_matmul=_FUSE_TLHS,
        ),
    )(x, dy)


# device time: 128366 ns/iter; 1.0474x vs baseline; 1.0474x over previous
import functools
import os

import jax
import jax.numpy as jnp
from jax import lax
from jax.experimental import pallas as pl
from jax.experimental.pallas import tpu as pltpu

MESH = pl.DeviceIdType.MESH
_COMPUTE_ONLY = bool(int(os.environ.get("KERNEL_COMPUTE_ONLY", "0")))
_NO_CID = bool(int(os.environ.get("KERNEL_NO_CID", "0")))
_FUSE_TLHS = bool(int(os.environ.get("KERNEL_FUSE_TLHS", "0")))
_DIAG = bool(int(os.environ.get("KERNEL_DIAG", "0")))


def kernel(x, dy):
    K, D = x.shape
    _, F = dy.shape
    G = F // 4
    H = D // 2
    T = 4
    TW = G // T
    HW = TW // 2
    U = 2 * T

    def body(x_hbm, dy_hbm, out_hbm, ld, xb, dyb, pk, zs, zr, sb, gx, gy,
             hx, hy, hd, cb, ld_sem, zs_s, zr_s, axs, axr, ays, ayr,
             bxs, bxr, bys, byr, ads, adr, sts, stc):
        mx = lax.axis_index("x")
        my = lax.axis_index("y")
        mz = lax.axis_index("z")
        g = 2 * mx + my
        gp = 2 * (1 - mx) + my
        hh = 2 * mx + (1 - my)
        hp = 2 * (1 - mx) + (1 - my)
        xpeer = (1 - mx, my, mz)
        ypeer = (mx, 1 - my, mz)
        zpeer = (mx, my, 1 - mz)
        dpeer = (1 - mx, 1 - my, mz)
        half0 = (1 - mz) * H
        half1 = mz * H

        load_plan = [("x", half0), ("x", half0 + TW), ("dy", 0), ("dy", 1),
                     ("x", half1), ("x", half1 + TW), ("dy", 2), ("dy", 3)]

        def start_load(c):
            kind, col = load_plan[c]
            src_col = col if kind == "x" else g * G + col * TW
            cp = pltpu.make_async_copy(
                (x_hbm if kind == "x" else dy_hbm).at[:, pl.ds(src_col, TW)],
                ld.at[c % 2], ld_sem.at[c % 2])
            cp.start()
            return cp

        pending = {0: start_load(0)}

        def finish_load(c):
            pending.pop(c).wait()
            if c + 1 < len(load_plan):
                pending[c + 1] = start_load(c + 1)
            kind, col = load_plan[c]
            val = ld[c % 2].astype(jnp.bfloat16)
            if kind == "x":
                xb[:, pl.ds(col, TW)] = val
            else:
                dyb[col % 2] = val

        if not _COMPUTE_ONLY and not _NO_CID:
            bar = pltpu.get_barrier_semaphore()
            nbrs = (xpeer, ypeer, zpeer) + ((dpeer,) if _DIAG else ())
            for dev in nbrs:
                pl.semaphore_signal(bar, inc=1, device_id=dev,
                                    device_id_type=MESH)
            pl.semaphore_wait(bar, len(nbrs))

        dn = (((0,), (0,)), ((), ()))

        def send_mm(t):
            p = lax.dot_general(
                xb[:, pl.ds(half0, H)], dyb[t % 2], dn,
                preferred_element_type=jnp.float32).astype(jnp.bfloat16)
            rs = []
            for u in (2 * t, 2 * t + 1):
                zs[u] = p[:, (u % 2) * HW:(u % 2) * HW + HW]
                if not _COMPUTE_ONLY:
                    r = pltpu.make_async_remote_copy(
                        zs.at[u], zr.at[u], zs_s.at[u], zr_s.at[u],
                        device_id=zpeer, device_id_type=MESH)
                    r.start()
                    rs.append(r)
            return rs

        def keep_mm(t):
            pk[t] = lax.dot_general(
                xb[:, pl.ds(half1, H)], dyb[t % 2], dn,
                preferred_element_type=jnp.float32)

        zrd = {}
        finish_load(0)
        finish_load(1)
        finish_load(2)
        zrd[0] = send_mm(0)
        finish_load(3)
        zrd[1] = send_mm(1)
        finish_load(4)
        finish_load(5)
        keep_mm(0)
        keep_mm(1)
        finish_load(6)
        zrd[2] = send_mm(2)
        keep_mm(2)
        finish_load(7)
        zrd[3] = send_mm(3)
        keep_mm(3)

        stcp = [None, None]
        cb_uses = [0]

        def store_via_cb(val_bf16, out_col):
            slot = cb_uses[0] % 2
            if stcp[slot] is not None:
                stcp[slot].wait()
            cb[slot] = val_bf16.astype(jnp.float32)
            cp = pltpu.make_async_copy(
                cb.at[slot], out_hbm.at[:, pl.ds(out_col, TW)], stc.at[slot])
            cp.start()
            stcp[slot] = cp
            cb_uses[0] += 1

        axd, ayd, std = {}, {}, []
        for u in range(U):
            t, i = u // 2, u % 2
            if not _COMPUTE_ONLY:
                zrd[t][i].wait()
            s = pk[t][:, i * HW:i * HW + HW] + (
                (zs if _COMPUTE_ONLY else zr)[u].astype(jnp.float32))
            pk[t, :, i * HW:i * HW + HW] = s
            sb[u] = s.astype(jnp.bfloat16)
            if not _COMPUTE_ONLY:
                ax = pltpu.make_async_remote_copy(
                    sb.at[u], gx.at[u], axs.at[u], axr.at[u],
                    device_id=xpeer, device_id_type=MESH)
                ax.start()
                axd[u] = ax
                ay = pltpu.make_async_remote_copy(
                    sb.at[u], gy.at[u], ays.at[u], ayr.at[u],
                    device_id=ypeer, device_id_type=MESH)
                ay.start()
                ayd[u] = ay
                if _DIAG and u == U - 1:
                    ad = pltpu.make_async_remote_copy(
                        sb.at[pl.ds(U - 2, 2)], hd, ads, adr,
                        device_id=dpeer, device_id_type=MESH)
                    ad.start()
            if i == 1:
                st = pltpu.make_async_copy(
                    pk.at[t], out_hbm.at[:, pl.ds(g * G + t * TW, TW)],
                    sts.at[t])
                st.start()
                std.append(st)

        bxd, byd = [], []
        gxr = sb if _COMPUTE_ONLY else gx
        gyr = sb if _COMPUTE_ONLY else gy
        for t in range(T):
            if not _COMPUTE_ONLY:
                axd[2 * t + 1].wait()
                if t < T - 1 or not _DIAG:
                    by = pltpu.make_async_remote_copy(
                        gx.at[2 * t + 1], hy.at[t], bys.at[t], byr.at[t],
                        device_id=ypeer, device_id_type=MESH)
                    by.start()
                    byd.append(by)
                ayd[2 * t].wait()
                if t < T - 1 or not _DIAG:
                    bx = pltpu.make_async_remote_copy(
                        gy.at[2 * t], hx.at[t], bxs.at[t], bxr.at[t],
                        device_id=xpeer, device_id_type=MESH)
                    bx.start()
                    bxd.append(bx)
                axd[2 * t].wait()
                ayd[2 * t + 1].wait()
            store_via_cb(
                jnp.concatenate([gxr[2 * t], gxr[2 * t + 1]], axis=1),
                gp * G + t * TW)
            store_via_cb(
                jnp.concatenate([gyr[2 * t], gyr[2 * t + 1]], axis=1),
                hh * G + t * TW)

        for t in range(T):
            if _COMPUTE_ONLY:
                hx_t, hy_t = sb[2 * t], sb[2 * t + 1]
            elif t < T - 1 or not _DIAG:
                bxd[t].wait()
                byd[t].wait()
                hx_t, hy_t = hx[t], hy[t]
            else:
                ad.wait()
                hx_t, hy_t = hd[0], hd[1]
            store_via_cb(
                jnp.concatenate([hx_t, hy_t], axis=1), hp * G + t * TW)

        for st in std:
            st.wait()
        for cp in stcp:
            if cp is not None:
                cp.wait()

        if not _COMPUTE_ONLY:
            @functools.partial(pl.run_scoped,
                               sem2=pltpu.SemaphoreType.REGULAR)
            def _(sem2):
                for dev in nbrs:
                    pl.semaphore_signal(sem2, inc=1, device_id=dev,
                                        device_id_type=MESH)
                pl.semaphore_wait(sem2, len(nbrs))

    return pl.pallas_call(
        body,
        out_shape=jax.ShapeDtypeStruct((H, F), jnp.float32),
        in_specs=[
            pl.BlockSpec(memory_space=pl.ANY),
            pl.BlockSpec(memory_space=pl.ANY),
        ],
        out_specs=pl.BlockSpec(memory_space=pl.ANY),
        scratch_shapes=[
            pltpu.VMEM((2, K, TW), jnp.float32),
            pltpu.VMEM((K, D), jnp.bfloat16),
            pltpu.VMEM((2, K, TW), jnp.bfloat16),
            pltpu.VMEM((T, H, TW), jnp.float32),
            pltpu.VMEM((U, H, HW), jnp.bfloat16),
            pltpu.VMEM((U, H, HW), jnp.bfloat16),
            pltpu.VMEM((U, H, HW), jnp.bfloat16),
            pltpu.VMEM((U, H, HW), jnp.bfloat16),
            pltpu.VMEM((U, H, HW), jnp.bfloat16),
            pltpu.VMEM((T, H, HW), jnp.bfloat16),
            pltpu.VMEM((T, H, HW), jnp.bfloat16),
            pltpu.VMEM((2, H, HW), jnp.bfloat16),
            pltpu.VMEM((2, H, TW), jnp.float32),
            pltpu.SemaphoreType.DMA((2,)),
            pltpu.SemaphoreType.DMA((U,)),
            pltpu.SemaphoreType.DMA((U,)),
            pltpu.SemaphoreType.DMA((U,)),
            pltpu.SemaphoreType.DMA((U,)),
            pltpu.SemaphoreType.DMA((U,)),
            pltpu.SemaphoreType.DMA((U,)),
            pltpu.SemaphoreType.DMA((T,)),
            pltpu.SemaphoreType.DMA((T,)),
            pltpu.SemaphoreType.DMA((T,)),
            pltpu.SemaphoreType.DMA((T,)),
            pltpu.SemaphoreType.DMA,
            pltpu.SemaphoreType.DMA,
            pltpu.SemaphoreType.DMA((T,)),
            pltpu.SemaphoreType.DMA((2,)),
        ],
        compiler_params=pltpu.CompilerParams(
            collective_id=None if (_COMPUTE_ONLY or _NO_CID) else 0,
            vmem_limit_bytes=100 * 1024 * 1024,
            fuse_transposed_lhs_in_matmul=_FUSE_TLHS,
        ),
    )(x, dy)


# device time: 121527 ns/iter; 1.1063x vs baseline; 1.0563x over previous
import functools
import os

import jax
import jax.numpy as jnp
from jax import lax
from jax.experimental import pallas as pl
from jax.experimental.pallas import tpu as pltpu

MESH = pl.DeviceIdType.MESH
_COMPUTE_ONLY = bool(int(os.environ.get("KERNEL_COMPUTE_ONLY", "0")))
_NO_CID = bool(int(os.environ.get("KERNEL_NO_CID", "0")))
_FUSE_TLHS = bool(int(os.environ.get("KERNEL_FUSE_TLHS", "0")))
_DIAG = bool(int(os.environ.get("KERNEL_DIAG", "0")))


def kernel(x, dy):
    K, D = x.shape
    _, F = dy.shape
    G = F // 4
    H = D // 2
    T = 4
    TW = G // T
    HW = TW // 2
    U = 2 * T

    def body(x_hbm, dy_hbm, out_hbm, ld, xb, dyb, pk, zs, zr, sb, gx, gy,
             hx, hy, hd, ld_sem, zs_s, zr_s, axs, axr, ays, ayr,
             bxs, bxr, bys, byr, ads, adr, sts):
        mx = lax.axis_index("x")
        my = lax.axis_index("y")
        mz = lax.axis_index("z")
        g = 2 * mx + my
        gp = 2 * (1 - mx) + my
        hh = 2 * mx + (1 - my)
        hp = 2 * (1 - mx) + (1 - my)
        xpeer = (1 - mx, my, mz)
        ypeer = (mx, 1 - my, mz)
        zpeer = (mx, my, 1 - mz)
        dpeer = (1 - mx, 1 - my, mz)
        half0 = (1 - mz) * H
        half1 = mz * H

        load_plan = [("x", half0), ("x", half0 + TW), ("dy", 0), ("dy", 1),
                     ("x", half1), ("x", half1 + TW), ("dy", 2), ("dy", 3)]

        def start_load(c):
            kind, col = load_plan[c]
            src_col = col if kind == "x" else g * G + col * TW
            cp = pltpu.make_async_copy(
                (x_hbm if kind == "x" else dy_hbm).at[:, pl.ds(src_col, TW)],
                ld.at[c % 2], ld_sem.at[c % 2])
            cp.start()
            return cp

        pending = {0: start_load(0)}

        def finish_load(c):
            pending.pop(c).wait()
            if c + 1 < len(load_plan):
                pending[c + 1] = start_load(c + 1)
            kind, col = load_plan[c]
            val = ld[c % 2].astype(jnp.bfloat16)
            if kind == "x":
                xb[:, pl.ds(col, TW)] = val
            else:
                dyb[col % 2] = val

        if not _COMPUTE_ONLY and not _NO_CID:
            bar = pltpu.get_barrier_semaphore()
            nbrs = (xpeer, ypeer, zpeer) + ((dpeer,) if _DIAG else ())
            for dev in nbrs:
                pl.semaphore_signal(bar, inc=1, device_id=dev,
                                    device_id_type=MESH)
            pl.semaphore_wait(bar, len(nbrs))

        dn = (((0,), (0,)), ((), ()))

        def send_mm(t):
            p = lax.dot_general(
                xb[:, pl.ds(half0, H)], dyb[t % 2], dn,
                preferred_element_type=jnp.float32).astype(jnp.bfloat16)
            rs = []
            for u in (2 * t, 2 * t + 1):
                zs[u] = p[:, (u % 2) * HW:(u % 2) * HW + HW]
                if not _COMPUTE_ONLY:
                    r = pltpu.make_async_remote_copy(
                        zs.at[u], zr.at[u], zs_s.at[u], zr_s.at[u],
                        device_id=zpeer, device_id_type=MESH)
                    r.start()
                    rs.append(r)
            return rs

        def keep_mm(t):
            pk[t] = lax.dot_general(
                xb[:, pl.ds(half1, H)], dyb[t % 2], dn,
                preferred_element_type=jnp.float32)

        zrd = {}
        finish_load(0)
        finish_load(1)
        finish_load(2)
        zrd[0] = send_mm(0)
        finish_load(3)
        zrd[1] = send_mm(1)
        finish_load(4)
        finish_load(5)
        keep_mm(0)
        keep_mm(1)
        finish_load(6)
        zrd[2] = send_mm(2)
        keep_mm(2)
        finish_load(7)
        zrd[3] = send_mm(3)
        keep_mm(3)

        st_cnt = [0]
        st_last = {}

        def store_direct(src, out_col):
            slot = st_cnt[0] % 8
            if slot in st_last:
                st_last[slot].wait()
            cp = pltpu.make_async_copy(
                src, out_hbm.at[:, pl.ds(out_col, HW)], sts.at[slot])
            cp.start()
            st_last[slot] = cp
            st_cnt[0] += 1

        axd, ayd, std = {}, {}, []
        for u in range(U):
            t, i = u // 2, u % 2
            if not _COMPUTE_ONLY:
                zrd[t][i].wait()
            s = pk[t][:, i * HW:i * HW + HW] + (
                (zs if _COMPUTE_ONLY else zr)[u].astype(jnp.float32))
            sb[u] = s.astype(jnp.bfloat16)
            store_direct(sb.at[u], g * G + u * HW)
            if not _COMPUTE_ONLY:
                ax = pltpu.make_async_remote_copy(
                    sb.at[u], gx.at[u], axs.at[u], axr.at[u],
                    device_id=xpeer, device_id_type=MESH)
                ax.start()
                axd[u] = ax
                ay = pltpu.make_async_remote_copy(
                    sb.at[u], gy.at[u], ays.at[u], ayr.at[u],
                    device_id=ypeer, device_id_type=MESH)
                ay.start()
                ayd[u] = ay
                if _DIAG and u == U - 1:
                    ad = pltpu.make_async_remote_copy(
                        sb.at[pl.ds(U - 2, 2)], hd, ads, adr,
                        device_id=dpeer, device_id_type=MESH)
                    ad.start()

        bxd, byd = [], []
        gxr = sb if _COMPUTE_ONLY else gx
        gyr = sb if _COMPUTE_ONLY else gy
        for t in range(T):
            if not _COMPUTE_ONLY:
                axd[2 * t + 1].wait()
                if t < T - 1 or not _DIAG:
                    by = pltpu.make_async_remote_copy(
                        gx.at[2 * t + 1], hy.at[t], bys.at[t], byr.at[t],
                        device_id=ypeer, device_id_type=MESH)
                    by.start()
                    byd.append(by)
                ayd[2 * t].wait()
                if t < T - 1 or not _DIAG:
                    bx = pltpu.make_async_remote_copy(
                        gy.at[2 * t], hx.at[t], bxs.at[t], bxr.at[t],
                        device_id=xpeer, device_id_type=MESH)
                    bx.start()
                    bxd.append(bx)
                axd[2 * t].wait()
                ayd[2 * t + 1].wait()
            store_direct(gxr.at[2 * t], gp * G + t * TW)
            store_direct(gxr.at[2 * t + 1], gp * G + t * TW + HW)
            store_direct(gyr.at[2 * t], hh * G + t * TW)
            store_direct(gyr.at[2 * t + 1], hh * G + t * TW + HW)

        for t in range(T):
            if _COMPUTE_ONLY:
                hx_t, hy_t = sb.at[2 * t], sb.at[2 * t + 1]
            elif t < T - 1 or not _DIAG:
                bxd[t].wait()
                byd[t].wait()
                hx_t, hy_t = hx.at[t], hy.at[t]
            else:
                ad.wait()
                hx_t, hy_t = hd.at[0], hd.at[1]
            store_direct(hx_t, hp * G + t * TW)
            store_direct(hy_t, hp * G + t * TW + HW)

        for cp in st_last.values():
            cp.wait()

        if not _COMPUTE_ONLY:
            @functools.partial(pl.run_scoped,
                               sem2=pltpu.SemaphoreType.REGULAR)
            def _(sem2):
                for dev in nbrs:
                    pl.semaphore_signal(sem2, inc=1, device_id=dev,
                                        device_id_type=MESH)
                pl.semaphore_wait(sem2, len(nbrs))

    def upcast(b):
        def ubody(b_ref, o_ref):
            o_ref[...] = b_ref[...].astype(jnp.float32)
        return pl.pallas_call(
            ubody,
            grid=(8,),
            out_shape=jax.ShapeDtypeStruct((H, F), jnp.float32),
            in_specs=[pl.BlockSpec((H, F // 8), lambda i: (0, i))],
            out_specs=pl.BlockSpec((H, F // 8), lambda i: (0, i)),
        )(b)

    gathered = pl.pallas_call(
        body,
        out_shape=jax.ShapeDtypeStruct((H, F), jnp.bfloat16),
        in_specs=[
            pl.BlockSpec(memory_space=pl.ANY),
            pl.BlockSpec(memory_space=pl.ANY),
        ],
        out_specs=pl.BlockSpec(memory_space=pl.ANY),
        scratch_shapes=[
            pltpu.VMEM((2, K, TW), jnp.float32),
            pltpu.VMEM((K, D), jnp.bfloat16),
            pltpu.VMEM((2, K, TW), jnp.bfloat16),
            pltpu.VMEM((T, H, TW), jnp.float32),
            pltpu.VMEM((U, H, HW), jnp.bfloat16),
            pltpu.VMEM((U, H, HW), jnp.bfloat16),
            pltpu.VMEM((U, H, HW), jnp.bfloat16),
            pltpu.VMEM((U, H, HW), jnp.bfloat16),
            pltpu.VMEM((U, H, HW), jnp.bfloat16),
            pltpu.VMEM((T, H, HW), jnp.bfloat16),
            pltpu.VMEM((T, H, HW), jnp.bfloat16),
            pltpu.VMEM((2, H, HW), jnp.bfloat16),
            pltpu.SemaphoreType.DMA((2,)),
            pltpu.SemaphoreType.DMA((U,)),
            pltpu.SemaphoreType.DMA((U,)),
            pltpu.SemaphoreType.DMA((U,)),
            pltpu.SemaphoreType.DMA((U,)),
            pltpu.SemaphoreType.DMA((U,)),
            pltpu.SemaphoreType.DMA((U,)),
            pltpu.SemaphoreType.DMA((T,)),
            pltpu.SemaphoreType.DMA((T,)),
            pltpu.SemaphoreType.DMA((T,)),
            pltpu.SemaphoreType.DMA((T,)),
            pltpu.SemaphoreType.DMA,
            pltpu.SemaphoreType.DMA,
            pltpu.SemaphoreType.DMA((8,)),
        ],
        compiler_params=pltpu.CompilerParams(
            collective_id=None if (_COMPUTE_ONLY or _NO_CID) else 0,
            vmem_limit_bytes=100 * 1024 * 1024,
            fuse_transposed_lhs_in_matmul=_FUSE_TLHS,
        ),
    )(x, dy)
    return upcast(gathered)


# device time: 121358 ns/iter; 1.1078x vs baseline; 1.0014x over previous
import functools
import os

import jax
import jax.numpy as jnp
from jax import lax
from jax.experimental import pallas as pl
from jax.experimental.pallas import tpu as pltpu

MESH = pl.DeviceIdType.MESH
_COMPUTE_ONLY = bool(int(os.environ.get("KERNEL_COMPUTE_ONLY", "0")))
_NO_CID = bool(int(os.environ.get("KERNEL_NO_CID", "0")))
_FUSE_TLHS = bool(int(os.environ.get("KERNEL_FUSE_TLHS", "0")))
_DIAG = bool(int(os.environ.get("KERNEL_DIAG", "0")))


def kernel(x, dy):
    K, D = x.shape
    _, F = dy.shape
    G = F // 4
    H = D // 2
    T = 4
    TW = G // T
    HW = TW // 2
    U = 2 * T

    def body(x_hbm, dy_hbm, out_hbm, ld, xb, dyb, pk, zs, zr, sb, gx, gy,
             hx, hy, hd, ld_sem, zs_s, zr_s, axs, axr, ays, ayr,
             bxs, bxr, bys, byr, ads, adr, sts):
        mx = lax.axis_index("x")
        my = lax.axis_index("y")
        mz = lax.axis_index("z")
        g = 2 * mx + my
        gp = 2 * (1 - mx) + my
        hh = 2 * mx + (1 - my)
        hp = 2 * (1 - mx) + (1 - my)
        xpeer = (1 - mx, my, mz)
        ypeer = (mx, 1 - my, mz)
        zpeer = (mx, my, 1 - mz)
        dpeer = (1 - mx, 1 - my, mz)
        half0 = (1 - mz) * H
        half1 = mz * H

        load_plan = [("x", half0), ("x", half0 + TW), ("dy", 0), ("dy", 1),
                     ("x", half1), ("x", half1 + TW), ("dy", 2), ("dy", 3)]

        def start_load(c):
            kind, col = load_plan[c]
            src_col = col if kind == "x" else g * G + col * TW
            cp = pltpu.make_async_copy(
                (x_hbm if kind == "x" else dy_hbm).at[:, pl.ds(src_col, TW)],
                ld.at[c % 2], ld_sem.at[c % 2])
            cp.start()
            return cp

        pending = {0: start_load(0)}

        def finish_load(c):
            pending.pop(c).wait()
            if c + 1 < len(load_plan):
                pending[c + 1] = start_load(c + 1)
            kind, col = load_plan[c]
            val = ld[c % 2].astype(jnp.bfloat16)
            if kind == "x":
                xb[:, pl.ds(col, TW)] = val
            else:
                dyb[col % 2] = val

        if not _COMPUTE_ONLY and not _NO_CID:
            bar = pltpu.get_barrier_semaphore()
            nbrs = (xpeer, ypeer, zpeer) + ((dpeer,) if _DIAG else ())
            for dev in nbrs:
                pl.semaphore_signal(bar, inc=1, device_id=dev,
                                    device_id_type=MESH)
            pl.semaphore_wait(bar, len(nbrs))

        dn = (((0,), (0,)), ((), ()))

        def send_mm(t):
            p = lax.dot_general(
                xb[:, pl.ds(half0, H)], dyb[t % 2], dn,
                preferred_element_type=jnp.float32).astype(jnp.bfloat16)
            rs = []
            for u in (2 * t, 2 * t + 1):
                zs[u] = p[:, (u % 2) * HW:(u % 2) * HW + HW]
                if not _COMPUTE_ONLY:
                    r = pltpu.make_async_remote_copy(
                        zs.at[u], zr.at[u], zs_s.at[u], zr_s.at[u],
                        device_id=zpeer, device_id_type=MESH)
                    r.start()
                    rs.append(r)
            return rs

        def keep_mm(t):
            pk[t] = lax.dot_general(
                xb[:, pl.ds(half1, H)], dyb[t % 2], dn,
                preferred_element_type=jnp.float32)

        zrd = {}
        finish_load(0)
        finish_load(1)
        finish_load(2)
        zrd[0] = send_mm(0)
        finish_load(3)
        zrd[1] = send_mm(1)
        finish_load(4)
        finish_load(5)
        keep_mm(0)
        keep_mm(1)
        finish_load(6)
        zrd[2] = send_mm(2)
        keep_mm(2)
        finish_load(7)
        zrd[3] = send_mm(3)
        keep_mm(3)

        st_cnt = [0]
        st_last = {}

        def store_direct(src, out_col):
            slot = st_cnt[0] % 8
            if slot in st_last:
                st_last[slot].wait()
            cp = pltpu.make_async_copy(
                src, out_hbm.at[:, pl.ds(out_col, HW)], sts.at[slot])
            cp.start()
            st_last[slot] = cp
            st_cnt[0] += 1

        axd, ayd = {}, {}
        for u in range(U):
            t, i = u // 2, u % 2
            if not _COMPUTE_ONLY:
                zrd[t][i].wait()
            s = pk[t][:, i * HW:i * HW + HW] + (
                (zs if _COMPUTE_ONLY else zr)[u].astype(jnp.float32))
            sb[u] = s.astype(jnp.bfloat16)
            store_direct(sb.at[u], g * G + u * HW)
            if not _COMPUTE_ONLY:
                ax = pltpu.make_async_remote_copy(
                    sb.at[u], gx.at[u], axs.at[u], axr.at[u],
                    device_id=xpeer, device_id_type=MESH)
                ax.start()
                axd[u] = ax
                ay = pltpu.make_async_remote_copy(
                    sb.at[u], gy.at[u], ays.at[u], ayr.at[u],
                    device_id=ypeer, device_id_type=MESH)
                ay.start()
                ayd[u] = ay
                if _DIAG and u == U - 1:
                    ad = pltpu.make_async_remote_copy(
                        sb.at[pl.ds(U - 2, 2)], hd, ads, adr,
                        device_id=dpeer, device_id_type=MESH)
                    ad.start()

        bxd, byd = [], []
        gxr = sb if _COMPUTE_ONLY else gx
        gyr = sb if _COMPUTE_ONLY else gy
        for t in range(T):
            if not _COMPUTE_ONLY:
                axd[2 * t + 1].wait()
                if t < T - 1 or not _DIAG:
                    by = pltpu.make_async_remote_copy(
                        gx.at[2 * t + 1], hy.at[t], bys.at[t], byr.at[t],
                        device_id=ypeer, device_id_type=MESH)
                    by.start()
                    byd.append(by)
                ayd[2 * t].wait()
                if t < T - 1 or not _DIAG:
                    bx = pltpu.make_async_remote_copy(
                        gy.at[2 * t], hx.at[t], bxs.at[t], bxr.at[t],
                        device_id=xpeer, device_id_type=MESH)
                    bx.start()
                    bxd.append(bx)
                axd[2 * t].wait()
                ayd[2 * t + 1].wait()
            store_direct(gxr.at[2 * t], gp * G + t * TW)
            store_direct(gxr.at[2 * t + 1], gp * G + t * TW + HW)
            store_direct(gyr.at[2 * t], hh * G + t * TW)
            store_direct(gyr.at[2 * t + 1], hh * G + t * TW + HW)

        for t in range(T):
            if _COMPUTE_ONLY:
                hx_t, hy_t = sb.at[2 * t], sb.at[2 * t + 1]
            elif t < T - 1 or not _DIAG:
                bxd[t].wait()
                byd[t].wait()
                hx_t, hy_t = hx.at[t], hy.at[t]
            else:
                ad.wait()
                hx_t, hy_t = hd.at[0], hd.at[1]
            store_direct(hx_t, hp * G + t * TW)
            store_direct(hy_t, hp * G + t * TW + HW)

        for cp in st_last.values():
            cp.wait()

        if not _COMPUTE_ONLY:
            @functools.partial(pl.run_scoped,
                               sem2=pltpu.SemaphoreType.REGULAR)
            def _(sem2):
                for dev in nbrs:
                    pl.semaphore_signal(sem2, inc=1, device_id=dev,
                                        device_id_type=MESH)
                pl.semaphore_wait(sem2, len(nbrs))

    def upcast(b):
        def ubody(b_ref, o_ref):
            o_ref[...] = b_ref[...].astype(jnp.float32)
        return pl.pallas_call(
            ubody,
            grid=(8,),
            out_shape=jax.ShapeDtypeStruct((H, F), jnp.float32),
            in_specs=[pl.BlockSpec((H, F // 8), lambda i: (0, i))],
            out_specs=pl.BlockSpec((H, F // 8), lambda i: (0, i)),
        )(b)

    gathered = pl.pallas_call(
        body,
        out_shape=jax.ShapeDtypeStruct((H, F), jnp.bfloat16),
        in_specs=[
            pl.BlockSpec(memory_space=pl.ANY),
            pl.BlockSpec(memory_space=pl.ANY),
        ],
        out_specs=pl.BlockSpec(memory_space=pl.ANY),
        scratch_shapes=[
            pltpu.VMEM((2, K, TW), jnp.float32),
            pltpu.VMEM((K, D), jnp.bfloat16),
            pltpu.VMEM((2, K, TW), jnp.bfloat16),
            pltpu.VMEM((T, H, TW), jnp.float32),
            pltpu.VMEM((U, H, HW), jnp.bfloat16),
            pltpu.VMEM((U, H, HW), jnp.bfloat16),
            pltpu.VMEM((U, H, HW), jnp.bfloat16),
            pltpu.VMEM((U, H, HW), jnp.bfloat16),
            pltpu.VMEM((U, H, HW), jnp.bfloat16),
            pltpu.VMEM((T, H, HW), jnp.bfloat16),
            pltpu.VMEM((T, H, HW), jnp.bfloat16),
            pltpu.VMEM((2, H, HW), jnp.bfloat16),
            pltpu.SemaphoreType.DMA((2,)),
            pltpu.SemaphoreType.DMA((U,)),
            pltpu.SemaphoreType.DMA((U,)),
            pltpu.SemaphoreType.DMA((U,)),
            pltpu.SemaphoreType.DMA((U,)),
            pltpu.SemaphoreType.DMA((U,)),
            pltpu.SemaphoreType.DMA((U,)),
            pltpu.SemaphoreType.DMA((T,)),
            pltpu.SemaphoreType.DMA((T,)),
            pltpu.SemaphoreType.DMA((T,)),
            pltpu.SemaphoreType.DMA((T,)),
            pltpu.SemaphoreType.DMA,
            pltpu.SemaphoreType.DMA,
            pltpu.SemaphoreType.DMA((8,)),
        ],
        compiler_params=pltpu.CompilerParams(
            collective_id=None if (_COMPUTE_ONLY or _NO_CID) else 0,
            vmem_limit_bytes=100 * 1024 * 1024,
            fuse_transposed_lhs_in_matmul=_FUSE_TLHS,
        ),
    )(x, dy)
    return upcast(gathered)
